# baseline (speedup 1.0000x reference)
import functools
import math

import jax
import jax.numpy as jnp
from jax import lax
from jax.experimental import pallas as pl
from jax.experimental.pallas import tpu as pltpu
from jax.experimental.pallas import tpu_sc as plsc

N_RBF = 50
CUTOFF = 5.0
RBF_PAD = 64
F = 128
LOG2 = math.log(2.0)

_WIDTH = CUTOFF / (N_RBF - 1)
_COEFF = -0.5 / _WIDTH ** 2


def _sp(x):
    return jnp.log1p(jnp.exp(jnp.minimum(x, 30.0))) + jnp.maximum(x, 30.0) - (30.0 + LOG2)


def _swish(x):
    return x * jax.nn.sigmoid(x)


E_PAD = 163840
N_PAD = 10240
NCG_PAD = 1024
_EPW = E_PAD // 32
_NPW = N_PAD // 32
_DCH = 64
_DNC = _EPW // _DCH
_RPT = N_PAD // 16


def _dsq_sc(xs, ys, zs, cgx, cgy, cgz, dst, src3d, mp):
    mesh = plsc.VectorSubcoreMesh(core_axis_name="c", subcore_axis_name="s")
    f32 = jnp.float32

    @functools.partial(
        pl.kernel, mesh=mesh,
        compiler_params=pltpu.CompilerParams(needs_layout_passes=False),
        out_type=[jax.ShapeDtypeStruct((E_PAD,), f32),
                  jax.ShapeDtypeStruct((N_PAD,), f32)],
        scratch_types=[
            pltpu.VMEM((N_PAD,), f32), pltpu.VMEM((N_PAD,), f32),
            pltpu.VMEM((N_PAD,), f32),
            pltpu.VMEM((NCG_PAD,), f32), pltpu.VMEM((NCG_PAD,), f32),
            pltpu.VMEM((NCG_PAD,), f32),
            pltpu.VMEM((_EPW,), jnp.int32),
            pltpu.VMEM((_EPW,), f32),
            pltpu.VMEM((_DNC, _DCH), jnp.int32),
            pltpu.VMEM((_NPW,), jnp.int32), pltpu.VMEM((_NPW,), f32),
        ])
    def body(xs_h, ys_h, zs_h, cgx_h, cgy_h, cgz_h, dst_h, s3_h, mp_h,
             oute_h, outa_h,
             xv, yv, zv, cxv, cyv, czv, db, eb, sb2, mb, ab):
        c = lax.axis_index("c")
        t = lax.axis_index("s")
        wid = t * 2 + c

        pltpu.sync_copy(xs_h, xv)
        pltpu.sync_copy(ys_h, yv)
        pltpu.sync_copy(zs_h, zv)
        pltpu.sync_copy(cgx_h, cxv)
        pltpu.sync_copy(cgy_h, cyv)
        pltpu.sync_copy(cgz_h, czv)
        ebase = wid * _EPW
        pltpu.sync_copy(dst_h.at[pl.ds(ebase, _EPW)], db)
        pltpu.sync_copy(s3_h.at[wid], sb2)

        def edge_it(r, _):
            for jj in range(_DCH // 16):
                si = sb2[r, pl.ds(jj * 16, 16)]
                sl = pl.ds(r * _DCH + jj * 16, 16)
                di = db[sl]
                dx = plsc.load_gather(xv, [di]) - plsc.load_gather(xv, [si])
                dy = plsc.load_gather(yv, [di]) - plsc.load_gather(yv, [si])
                dz = plsc.load_gather(zv, [di]) - plsc.load_gather(zv, [si])
                eb[sl] = dx * dx + dy * dy + dz * dz + 3e-8
            return 0

        lax.fori_loop(0, _DNC, edge_it, 0)
        pltpu.sync_copy(eb, oute_h.at[pl.ds(ebase, _EPW)])

        abase = wid * _NPW
        pltpu.sync_copy(mp_h.at[pl.ds(abase, _NPW)], mb)

        def atom_it(i, _):
            sl = pl.ds(i * 16, 16)
            asl = pl.ds(abase + i * 16, 16)
            mi = mb[sl]
            dx = xv[asl] - plsc.load_gather(cxv, [mi])
            dy = yv[asl] - plsc.load_gather(cyv, [mi])
            dz = zv[asl] - plsc.load_gather(czv, [mi])
            ab[sl] = dx * dx + dy * dy + dz * dz + 3e-8
            return 0

        lax.fori_loop(0, _NPW // 16, atom_it, 0)
        pltpu.sync_copy(ab, outa_h.at[pl.ds(abase, _NPW)])

    return body(xs, ys, zs, cgx, cgy, cgz, dst, src3d, mp)


def _deg_sc(src3d):
    mesh = plsc.VectorSubcoreMesh(core_axis_name="c", subcore_axis_name="s")
    f32 = jnp.float32

    @functools.partial(
        pl.kernel, mesh=mesh,
        compiler_params=pltpu.CompilerParams(needs_layout_passes=False),
        out_type=jax.ShapeDtypeStruct((2 * N_PAD,), f32),
        scratch_types=[
            pltpu.VMEM((_DNC, _DCH), jnp.int32),
            pltpu.VMEM_SHARED((N_PAD, F), f32),
            pltpu.VMEM((128, F), f32),
            pltpu.VMEM((_RPT,), f32),
        ])
    def body(s3_h, degp_h, sb2, dacc, vb, ext):
        c = lax.axis_index("c")
        t = lax.axis_index("s")
        wid = t * 2 + c

        def zrow(i, _):
            for j in range(F // 16):
                vb[i, pl.ds(j * 16, 16)] = jnp.zeros((16,), f32)
            return 0

        lax.fori_loop(0, 128, zrow, 0)
        for k in range(_RPT // 128):
            pltpu.sync_copy(vb, dacc.at[pl.ds(t * _RPT + k * 128, 128)])
        plsc.subcore_barrier()

        pltpu.sync_copy(s3_h.at[wid], sb2)

        def orow(i, _):
            for j in range(F // 16):
                vb[i, pl.ds(j * 16, 16)] = jnp.ones((16,), f32)
            return 0

        lax.fori_loop(0, _DCH, orow, 0)

        def deg_it(k, _):
            pltpu.sync_copy(vb.at[pl.ds(0, _DCH)], dacc.at[sb2.at[k]],
                            add=True)
            return 0

        lax.fori_loop(0, _DNC, deg_it, 0)
        plsc.subcore_barrier()

        zero16 = jnp.zeros((16,), jnp.int32)
        for grp in range(_RPT // 128):
            pltpu.sync_copy(dacc.at[pl.ds(t * _RPT + grp * 128, 128)], vb)
            for j in range(8):
                rows = lax.iota(jnp.int32, 16) + j * 16
                ext[pl.ds(grp * 128 + j * 16, 16)] = plsc.load_gather(
                    vb, [rows, zero16])
        pltpu.sync_copy(ext, degp_h.at[pl.ds(c * N_PAD + t * _RPT, _RPT)])

    return body(src3d)


_EPT = E_PAD // 32
_CHUNK = 128
_NCHUNK = _EPT // _CHUNK


def _scatter_sc(hval2, src_c):
    mesh = plsc.VectorSubcoreMesh(core_axis_name="c", subcore_axis_name="s")
    f32 = jnp.float32

    @functools.partial(
        pl.kernel, mesh=mesh,
        compiler_params=pltpu.CompilerParams(needs_layout_passes=False),
        out_type=jax.ShapeDtypeStruct((2, N_PAD, F), f32),
        scratch_types=[
            pltpu.VMEM_SHARED((N_PAD, F), f32),
            pltpu.VMEM((_CHUNK, F), f32), pltpu.VMEM((_CHUNK, F), f32),
            pltpu.VMEM((_CHUNK,), jnp.int32), pltpu.VMEM((_CHUNK,), jnp.int32),
            pltpu.SemaphoreType.DMA, pltpu.SemaphoreType.DMA,
        ])
    def body(hval_h, src_h, out_h, es, vb0, vb1, ib0, ib1, sem0, sem1):
        c = lax.axis_index("c")
        t = lax.axis_index("s")
        bufs = ((vb0, ib0, sem0), (vb1, ib1, sem1))

        def zrow(i, _):
            for j in range(F // 16):
                vb0[i, pl.ds(j * 16, 16)] = jnp.zeros((16,), f32)
            return 0

        lax.fori_loop(0, _CHUNK, zrow, 0)
        for k in range(_RPT // _CHUNK):
            pltpu.sync_copy(vb0, es.at[pl.ds(t * _RPT + k * _CHUNK, _CHUNK)])
        plsc.subcore_barrier()

        half = E_PAD // 2

        def start_load(k, vb, ib, sem):
            ebase = c * half + t * _EPT + k * _CHUNK
            pltpu.async_copy(src_h.at[pl.ds(ebase, _CHUNK)], ib, sem)
            pltpu.async_copy(hval_h.at[pl.ds(ebase, _CHUNK)], vb, sem)

        def wait_load(vb, ib, sem):
            pltpu.make_async_copy(src_h.at[pl.ds(0, _CHUNK)], ib, sem).wait()
            pltpu.make_async_copy(hval_h.at[pl.ds(0, _CHUNK)], vb, sem).wait()

        for b in range(2):
            vb, ib, sem = bufs[b]
            start_load(b, vb, ib, sem)

        def chunk_pair(kk, _):
            for b in range(2):
                k = kk * 2 + b
                vb, ib, sem = bufs[b]

                @pl.when(k < _NCHUNK)
                def _do():
                    wait_load(vb, ib, sem)
                    pltpu.sync_copy(vb, es.at[ib], add=True)

                    @pl.when(k + 2 < _NCHUNK)
                    def _next():
                        start_load(k + 2, vb, ib, sem)
            return 0

        lax.fori_loop(0, (_NCHUNK + 1) // 2, chunk_pair, 0)
        plsc.subcore_barrier()

        rbase = t * _RPT
        pltpu.sync_copy(es.at[pl.ds(rbase, _RPT)],
                        out_h.at[c, pl.ds(rbase, _RPT)])

    return body(hval2, src_c)


def _edge_kernel(dsq_ref, offs_ref, w1_ref, b1_ref, out_ref):
    d3 = jnp.sqrt(dsq_ref[:, :])[:, :, None]
    o3 = jnp.reshape(offs_ref[:, :], (1, 1, RBF_PAD))
    g3 = jnp.exp(_COEFF * (d3 - o3) ** 2)
    dn = (((2,), (0,)), ((), ()))
    b3 = jnp.reshape(b1_ref[:, :], (1, 1, F))
    out_ref[:, :, :] = _sp(lax.dot_general(
        g3, w1_ref[:, :], dn, preferred_element_type=jnp.float32) + b3)


def _edge_dense(dsq_pad, offs, w1, b1, e_pad, blk):
    grid = e_pad // blk
    rows = blk // 128
    return pl.pallas_call(
        _edge_kernel,
        grid=(grid,),
        in_specs=[
            pl.BlockSpec((rows, 128), lambda i: (i, 0)),
            pl.BlockSpec((1, RBF_PAD), lambda i: (0, 0)),
            pl.BlockSpec((RBF_PAD, F), lambda i: (0, 0)),
            pl.BlockSpec((1, F), lambda i: (0, 0)),
        ],
        out_specs=pl.BlockSpec((rows, 128, F), lambda i: (i, 0, 0)),
        out_shape=jax.ShapeDtypeStruct((e_pad // 128, 128, F), jnp.float32),
    )(dsq_pad, offs, w1, b1)


def _node_layer_kernel(ES_ref, z_ref, h_ref, f1cg_ref, m_ref, dsqI_ref, deg_ref,
                       embed_ref, offs_ref,
                       aw2, ab2, l1, l1b, l2, l2b, f1w, f1b,
                       cw1, cb1, cw2, cb2, cf1w, cf1b,
                       *rest, first, last):
    if last:
        out_h, out_f, out_ref, acc_ref = rest
    else:
        out_h, out_f = rest
    b = z_ref.shape[0]
    f32 = jnp.float32
    pid = pl.program_id(0)
    nprog = pl.num_programs(0)

    def dot(a, b_):
        return jnp.dot(a, b_, preferred_element_type=f32)

    if first:
        z = jnp.broadcast_to(z_ref[:, :], (b, F))
        onehot_z = (z == lax.broadcasted_iota(jnp.int32, (b, F), 1)).astype(f32)
        h = dot(onehot_z, embed_ref[:, :])
        f1cg_in = jnp.zeros((b, F), f32)
    else:
        h = h_ref[:, :]
        f1cg_in = f1cg_ref[:, :]

    dI = jnp.sqrt(dsqI_ref[:, :])
    deg = deg_ref[:, :]
    g_a = jnp.exp(_COEFF * (dI - offs_ref[:, :]) ** 2)

    ESL = ES_ref[0] + ES_ref[1]
    Esum = dot(ESL, aw2[:, :]) + deg * ab2[:, :]
    pre = dot(h, l1[:, :]) + l1b[:, :]
    phi = dot(_swish(pre), l2[:, :]) + l2b[:, :]
    h = 2.0 * h + dot(phi * Esum, f1w[:, :]) + deg * f1b[:, :]
    ecg = dot(_sp(dot(g_a, cw1[:, :]) + cb1[:, :]), cw2[:, :]) + cb2[:, :]
    f1cg = f1cg_in + dot(h * ecg, cf1w[:, :]) + cf1b[:, :]
    out_h[:, :] = h
    out_f[:, :] = f1cg

    if last:
        @pl.when(pid == 0)
        def _init():
            acc_ref[:, :] = jnp.zeros_like(acc_ref)

        ncg = out_ref.shape[0]
        ones_aux = (lax.broadcasted_iota(jnp.int32, (b, 8), 1) == 0).astype(f32)
        rhs = jnp.concatenate([f1cg, ones_aux], axis=1)
        mblk = jnp.broadcast_to(m_ref[:, :], (b, ncg))
        oh = (mblk == lax.broadcasted_iota(jnp.int32, (b, ncg), 1)).astype(f32)
        acc_ref[:, :] += lax.dot_general(
            oh, rhs, dimension_numbers=(((0,), (0,)), ((), ())),
            preferred_element_type=f32)

        @pl.when(pid == nprog - 1)
        def _fin():
            acc = acc_ref[:, :]
            cnt = jnp.maximum(acc[:, F:F + 1], 1.0)
            out_ref[:, :] = acc[:, :F] / cnt


def _node_layer(ES, z2, h_in, f1cg_in, m2, dsqI, deg2, embed_pad, offs,
                wlist, n, ncg, blk, first, last):
    grid = n // blk
    full = lambda shape: pl.BlockSpec(shape, lambda i: tuple(0 for _ in shape))
    wspecs = [full(w.shape) for w in wlist]
    col = pl.BlockSpec((blk, 1), lambda i: (i, 0))
    mat = pl.BlockSpec((blk, F), lambda i: (i, 0))
    out_shapes = [jax.ShapeDtypeStruct((n, F), jnp.float32),
                  jax.ShapeDtypeStruct((n, F), jnp.float32)]
    out_specs = [mat, mat]
    scratch = []
    if last:
        out_shapes.append(jax.ShapeDtypeStruct((ncg, F), jnp.float32))
        out_specs.append(pl.BlockSpec((ncg, F), lambda i: (0, 0)))
        scratch = [pltpu.VMEM((ncg, F + 8), jnp.float32)]
    return pl.pallas_call(
        functools.partial(_node_layer_kernel, first=first, last=last),
        grid=(grid,),
        in_specs=[
            pl.BlockSpec((2, blk, F), lambda i: (0, i, 0)),
            col, mat, mat, col, col, col,
            full((F, F)),
            full((1, RBF_PAD)),
        ] + wspecs,
        out_specs=out_specs,
        out_shape=out_shapes,
        scratch_shapes=scratch,
    )(ES, z2, h_in, f1cg_in, m2, dsqI, deg2, embed_pad, offs, *wlist)


def _pad_rows(w):
    return jnp.concatenate([w, jnp.zeros((RBF_PAD - w.shape[0], w.shape[1]), w.dtype)], axis=0)


def kernel(z, xyz, cg_xyz, mapping, nbr_list, atom_embed, params):
    n = xyz.shape[0]
    ncg = cg_xyz.shape[0]
    e = nbr_list.shape[0]
    e_pad = E_PAD

    z = z.astype(jnp.int32)
    mapping = mapping.astype(jnp.int32)
    nbr_list = nbr_list.astype(jnp.int32)

    src = nbr_list[:, 0]
    dst = nbr_list[:, 1]

    xyzf = xyz.astype(jnp.float32)
    cgf = cg_xyz.astype(jnp.float32)
    zero_n = jnp.zeros((N_PAD - n,), jnp.float32)
    xs = jnp.concatenate([xyzf[:, 0], zero_n])
    ys = jnp.concatenate([xyzf[:, 1], zero_n])
    zs = jnp.concatenate([xyzf[:, 2], zero_n])
    zero_c = jnp.zeros((NCG_PAD - ncg,), jnp.float32)
    cgx = jnp.concatenate([cgf[:, 0], zero_c])
    cgy = jnp.concatenate([cgf[:, 1], zero_c])
    cgz = jnp.concatenate([cgf[:, 2], zero_c])
    pad_e0 = jnp.zeros((e_pad - e,), jnp.int32)
    dst_a = jnp.concatenate([dst, pad_e0])
    src_c = jnp.concatenate([src, jnp.full((e_pad - e,), n, jnp.int32)])
    src3d = src_c.reshape(32, _DNC, _DCH)
    mp_pad = jnp.concatenate([mapping, jnp.zeros((N_PAD - n,), jnp.int32)])

    dsq_e, dsq_a = _dsq_sc(xs, ys, zs, cgx, cgy, cgz, dst_a, src3d, mp_pad)
    degp = _deg_sc(src3d)
    dsq_pad = dsq_e.reshape(e_pad // 128, 128)
    deg = (degp[:N_PAD] + degp[N_PAD:])[:n]

    offs = jnp.concatenate([jnp.linspace(0.0, CUTOFF, N_RBF, dtype=jnp.float32),
                            jnp.zeros((RBF_PAD - N_RBF,), jnp.float32)]).reshape(1, RBF_PAD)

    pa0, pa1 = params['atom'][0], params['atom'][1]
    hv0 = _edge_dense(dsq_pad, offs, _pad_rows(pa0['dist_w1']),
                      pa0['dist_b1'].reshape(1, F), e_pad, 8192)
    ES0 = _scatter_sc(hv0.reshape(e_pad, F), src_c)
    hv1 = _edge_dense(dsq_pad, offs, _pad_rows(pa1['dist_w1']),
                      pa1['dist_b1'].reshape(1, F), e_pad, 8192)
    ES1 = _scatter_sc(hv1.reshape(e_pad, F), src_c)

    z2 = z.reshape(n, 1)
    m2 = mapping.reshape(n, 1)
    dsqI2 = dsq_a[:n].reshape(n, 1).astype(jnp.float32)
    deg2 = deg.reshape(n, 1)
    embed_pad = jnp.concatenate(
        [atom_embed, jnp.zeros((F - atom_embed.shape[0], F), jnp.float32)], axis=0)

    wl = []
    for L in range(2):
        pa = params['atom'][L]
        pc = params['cg'][L]
        wl.append([pa['dist_w2'], pa['dist_b2'].reshape(1, F),
                   pa['l1_w'], pa['l1_b'].reshape(1, F),
                   pa['l2_w'], pa['l2_b'].reshape(1, F),
                   pa['f1_w'], pa['f1_b'].reshape(1, F),
                   _pad_rows(pc['dist_w1']), pc['dist_b1'].reshape(1, F),
                   pc['dist_w2'], pc['dist_b2'].reshape(1, F),
                   pc['f1_w'], pc['f1_b'].reshape(1, F)])

    dummy = jnp.zeros((n, F), jnp.float32)
    h1, f1cg0 = _node_layer(ES0, z2, dummy, dummy, m2, dsqI2, deg2, embed_pad,
                            offs, wl[0], n, ncg, 2000, first=True, last=False)
    _, _, H = _node_layer(ES1, z2, h1, f1cg0, m2, dsqI2, deg2, embed_pad,
                          offs, wl[1], n, ncg, 2000, first=False, last=True)
    return H


# speedup vs baseline: 1.0077x; 1.0077x over previous
import functools
import math

import jax
import jax.numpy as jnp
from jax import lax
from jax.experimental import pallas as pl
from jax.experimental.pallas import tpu as pltpu
from jax.experimental.pallas import tpu_sc as plsc

N_RBF = 50
CUTOFF = 5.0
RBF_PAD = 64
F = 128
LOG2 = math.log(2.0)

_WIDTH = CUTOFF / (N_RBF - 1)
_COEFF = -0.5 / _WIDTH ** 2


def _sp(x):
    return jnp.log1p(jnp.exp(jnp.minimum(x, 30.0))) + jnp.maximum(x, 30.0) - (30.0 + LOG2)


def _swish(x):
    return x * jax.nn.sigmoid(x)


E_PAD = 163840
N_PAD = 10240
NCG_PAD = 1024
_EPW = E_PAD // 32
_NPW = N_PAD // 32
_DCH = 64
_DNC = _EPW // _DCH
_RPT = N_PAD // 16


def _dsq_sc(xs, ys, zs, cgx, cgy, cgz, dst, src3d, mp):
    mesh = plsc.VectorSubcoreMesh(core_axis_name="c", subcore_axis_name="s")
    f32 = jnp.float32

    @functools.partial(
        pl.kernel, mesh=mesh,
        compiler_params=pltpu.CompilerParams(needs_layout_passes=False),
        out_type=[jax.ShapeDtypeStruct((E_PAD,), f32),
                  jax.ShapeDtypeStruct((N_PAD,), f32)],
        scratch_types=[
            pltpu.VMEM((N_PAD,), f32), pltpu.VMEM((N_PAD,), f32),
            pltpu.VMEM((N_PAD,), f32),
            pltpu.VMEM((NCG_PAD,), f32), pltpu.VMEM((NCG_PAD,), f32),
            pltpu.VMEM((NCG_PAD,), f32),
            pltpu.VMEM((_EPW,), jnp.int32),
            pltpu.VMEM((_EPW,), f32),
            pltpu.VMEM((_DNC, _DCH), jnp.int32),
            pltpu.VMEM((_NPW,), jnp.int32), pltpu.VMEM((_NPW,), f32),
        ])
    def body(xs_h, ys_h, zs_h, cgx_h, cgy_h, cgz_h, dst_h, s3_h, mp_h,
             oute_h, outa_h,
             xv, yv, zv, cxv, cyv, czv, db, eb, sb2, mb, ab):
        c = lax.axis_index("c")
        t = lax.axis_index("s")
        wid = t * 2 + c

        pltpu.sync_copy(xs_h, xv)
        pltpu.sync_copy(ys_h, yv)
        pltpu.sync_copy(zs_h, zv)
        pltpu.sync_copy(cgx_h, cxv)
        pltpu.sync_copy(cgy_h, cyv)
        pltpu.sync_copy(cgz_h, czv)
        ebase = wid * _EPW
        pltpu.sync_copy(dst_h.at[pl.ds(ebase, _EPW)], db)
        pltpu.sync_copy(s3_h.at[wid], sb2)

        def edge_it(r, _):
            for jj in range(_DCH // 16):
                si = sb2[r, pl.ds(jj * 16, 16)]
                sl = pl.ds(r * _DCH + jj * 16, 16)
                di = db[sl]
                dx = plsc.load_gather(xv, [di]) - plsc.load_gather(xv, [si])
                dy = plsc.load_gather(yv, [di]) - plsc.load_gather(yv, [si])
                dz = plsc.load_gather(zv, [di]) - plsc.load_gather(zv, [si])
                eb[sl] = dx * dx + dy * dy + dz * dz + 3e-8
            return 0

        lax.fori_loop(0, _DNC, edge_it, 0)
        pltpu.sync_copy(eb, oute_h.at[pl.ds(ebase, _EPW)])

        abase = wid * _NPW
        pltpu.sync_copy(mp_h.at[pl.ds(abase, _NPW)], mb)

        def atom_it(i, _):
            sl = pl.ds(i * 16, 16)
            asl = pl.ds(abase + i * 16, 16)
            mi = mb[sl]
            dx = xv[asl] - plsc.load_gather(cxv, [mi])
            dy = yv[asl] - plsc.load_gather(cyv, [mi])
            dz = zv[asl] - plsc.load_gather(czv, [mi])
            ab[sl] = dx * dx + dy * dy + dz * dz + 3e-8
            return 0

        lax.fori_loop(0, _NPW // 16, atom_it, 0)
        pltpu.sync_copy(ab, outa_h.at[pl.ds(abase, _NPW)])

    return body(xs, ys, zs, cgx, cgy, cgz, dst, src3d, mp)


def _deg_sc(src3d):
    mesh = plsc.VectorSubcoreMesh(core_axis_name="c", subcore_axis_name="s")
    f32 = jnp.float32

    @functools.partial(
        pl.kernel, mesh=mesh,
        compiler_params=pltpu.CompilerParams(needs_layout_passes=False),
        out_type=jax.ShapeDtypeStruct((2 * N_PAD,), f32),
        scratch_types=[
            pltpu.VMEM((_DNC, _DCH), jnp.int32),
            pltpu.VMEM_SHARED((N_PAD, F), f32),
            pltpu.VMEM((128, F), f32),
            pltpu.VMEM((_RPT,), f32),
        ])
    def body(s3_h, degp_h, sb2, dacc, vb, ext):
        c = lax.axis_index("c")
        t = lax.axis_index("s")
        wid = t * 2 + c

        def zrow(i, _):
            for j in range(F // 16):
                vb[i, pl.ds(j * 16, 16)] = jnp.zeros((16,), f32)
            return 0

        lax.fori_loop(0, 128, zrow, 0)
        for k in range(_RPT // 128):
            pltpu.sync_copy(vb, dacc.at[pl.ds(t * _RPT + k * 128, 128)])
        plsc.subcore_barrier()

        pltpu.sync_copy(s3_h.at[wid], sb2)

        def orow(i, _):
            for j in range(F // 16):
                vb[i, pl.ds(j * 16, 16)] = jnp.ones((16,), f32)
            return 0

        lax.fori_loop(0, _DCH, orow, 0)

        def deg_it(k, _):
            pltpu.sync_copy(vb.at[pl.ds(0, _DCH)], dacc.at[sb2.at[k]],
                            add=True)
            return 0

        lax.fori_loop(0, _DNC, deg_it, 0)
        plsc.subcore_barrier()

        zero16 = jnp.zeros((16,), jnp.int32)
        for grp in range(_RPT // 128):
            pltpu.sync_copy(dacc.at[pl.ds(t * _RPT + grp * 128, 128)], vb)
            for j in range(8):
                rows = lax.iota(jnp.int32, 16) + j * 16
                ext[pl.ds(grp * 128 + j * 16, 16)] = plsc.load_gather(
                    vb, [rows, zero16])
        pltpu.sync_copy(ext, degp_h.at[pl.ds(c * N_PAD + t * _RPT, _RPT)])

    return body(src3d)


_EPT = E_PAD // 32
_CHUNK = 128
_NCHUNK = _EPT // _CHUNK


def _scatter_sc(hval2, src_c):
    mesh = plsc.VectorSubcoreMesh(core_axis_name="c", subcore_axis_name="s")
    f32 = jnp.float32

    @functools.partial(
        pl.kernel, mesh=mesh,
        compiler_params=pltpu.CompilerParams(needs_layout_passes=False),
        out_type=jax.ShapeDtypeStruct((2, N_PAD, F), f32),
        scratch_types=[
            pltpu.VMEM_SHARED((N_PAD, F), f32),
            pltpu.VMEM((_CHUNK, F), f32), pltpu.VMEM((_CHUNK, F), f32),
            pltpu.VMEM((_CHUNK,), jnp.int32), pltpu.VMEM((_CHUNK,), jnp.int32),
            pltpu.SemaphoreType.DMA, pltpu.SemaphoreType.DMA,
        ])
    def body(hval_h, src_h, out_h, es, vb0, vb1, ib0, ib1, sem0, sem1):
        c = lax.axis_index("c")
        t = lax.axis_index("s")
        bufs = ((vb0, ib0, sem0), (vb1, ib1, sem1))

        def zrow(i, _):
            for j in range(F // 16):
                vb0[i, pl.ds(j * 16, 16)] = jnp.zeros((16,), f32)
            return 0

        lax.fori_loop(0, _CHUNK, zrow, 0)
        for k in range(_RPT // _CHUNK):
            pltpu.sync_copy(vb0, es.at[pl.ds(t * _RPT + k * _CHUNK, _CHUNK)])
        plsc.subcore_barrier()

        half = E_PAD // 2

        def start_load(k, vb, ib, sem):
            ebase = c * half + t * _EPT + k * _CHUNK
            pltpu.async_copy(src_h.at[pl.ds(ebase, _CHUNK)], ib, sem)
            pltpu.async_copy(hval_h.at[pl.ds(ebase, _CHUNK)], vb, sem)

        def wait_load(vb, ib, sem):
            pltpu.make_async_copy(src_h.at[pl.ds(0, _CHUNK)], ib, sem).wait()
            pltpu.make_async_copy(hval_h.at[pl.ds(0, _CHUNK)], vb, sem).wait()

        for b in range(2):
            vb, ib, sem = bufs[b]
            start_load(b, vb, ib, sem)

        def chunk_pair(kk, _):
            for b in range(2):
                k = kk * 2 + b
                vb, ib, sem = bufs[b]

                @pl.when(k < _NCHUNK)
                def _do():
                    wait_load(vb, ib, sem)
                    pltpu.sync_copy(vb, es.at[ib], add=True)

                    @pl.when(k + 2 < _NCHUNK)
                    def _next():
                        start_load(k + 2, vb, ib, sem)
            return 0

        lax.fori_loop(0, (_NCHUNK + 1) // 2, chunk_pair, 0)
        plsc.subcore_barrier()

        rbase = t * _RPT
        pltpu.sync_copy(es.at[pl.ds(rbase, _RPT)],
                        out_h.at[c, pl.ds(rbase, _RPT)])

    return body(hval2, src_c)


def _edge_kernel(dsq_ref, offs_ref, w1_ref, b1_ref, out_ref):
    d3 = jnp.sqrt(dsq_ref[:, :])[:, :, None]
    o3 = jnp.reshape(offs_ref[:, :], (1, 1, RBF_PAD))
    g3 = jnp.exp(_COEFF * (d3 - o3) ** 2)
    dn = (((2,), (0,)), ((), ()))
    b3 = jnp.reshape(b1_ref[:, :], (1, 1, F))
    out_ref[:, :, :] = _sp(lax.dot_general(
        g3, w1_ref[:, :], dn, preferred_element_type=jnp.float32) + b3)


def _edge_dense(dsq_pad, offs, w1, b1, e_pad, blk):
    grid = e_pad // blk
    rows = blk // 128
    return pl.pallas_call(
        _edge_kernel,
        grid=(grid,),
        in_specs=[
            pl.BlockSpec((rows, 128), lambda i: (i, 0)),
            pl.BlockSpec((1, RBF_PAD), lambda i: (0, 0)),
            pl.BlockSpec((RBF_PAD, F), lambda i: (0, 0)),
            pl.BlockSpec((1, F), lambda i: (0, 0)),
        ],
        out_specs=pl.BlockSpec((rows, 128, F), lambda i: (i, 0, 0)),
        out_shape=jax.ShapeDtypeStruct((e_pad // 128, 128, F), jnp.float32),
    )(dsq_pad, offs, w1, b1)


def _node_kernel(ES0_ref, ES1_ref, z_ref, m_ref, dsqI_ref, deg_ref, embed_ref, offs_ref,
                 aw2_0, ab2_0, l1_0, l1b_0, l2_0, l2b_0, f1w_0, f1b_0,
                 cw1_0, cb1_0, cw2_0, cb2_0, cf1w_0, cf1b_0,
                 aw2_1, ab2_1, l1_1, l1b_1, l2_1, l2b_1, f1w_1, f1b_1,
                 cw1_1, cb1_1, cw2_1, cb2_1, cf1w_1, cf1b_1,
                 out_ref, acc_ref):
    b = z_ref.shape[0]
    ncg = out_ref.shape[0]
    f32 = jnp.float32
    pid = pl.program_id(0)
    nprog = pl.num_programs(0)

    @pl.when(pid == 0)
    def _init():
        acc_ref[:, :] = jnp.zeros_like(acc_ref)

    def dot(a, b_):
        return jnp.dot(a, b_, preferred_element_type=f32)

    z = jnp.broadcast_to(z_ref[:, :], (b, F))
    onehot_z = (z == lax.broadcasted_iota(jnp.int32, (b, F), 1)).astype(f32)
    h = dot(onehot_z, embed_ref[:, :])

    dI = jnp.sqrt(dsqI_ref[:, :])
    deg = deg_ref[:, :]
    g_a = jnp.exp(_COEFF * (dI - offs_ref[:, :]) ** 2)

    aw = ((aw2_0, ab2_0, l1_0, l1b_0, l2_0, l2b_0, f1w_0, f1b_0),
          (aw2_1, ab2_1, l1_1, l1b_1, l2_1, l2b_1, f1w_1, f1b_1))
    cw = ((cw1_0, cb1_0, cw2_0, cb2_0, cf1w_0, cf1b_0),
          (cw1_1, cb1_1, cw2_1, cb2_1, cf1w_1, cf1b_1))

    f1cg_tot = jnp.zeros((b, F), f32)
    for L in range(2):
        aw2, ab2, l1, l1b, l2, l2b, f1w, f1b = aw[L]
        cw1, cb1, cw2, cb2, cf1w, cf1b = cw[L]
        ES_r = ES0_ref if L == 0 else ES1_ref
        ESL = ES_r[0] + ES_r[1]
        Esum = dot(ESL, aw2[:, :]) + deg * ab2[:, :]
        pre = dot(h, l1[:, :]) + l1b[:, :]
        phi = dot(_swish(pre), l2[:, :]) + l2b[:, :]
        h = 2.0 * h + dot(phi * Esum, f1w[:, :]) + deg * f1b[:, :]
        ecg = dot(_sp(dot(g_a, cw1[:, :]) + cb1[:, :]), cw2[:, :]) + cb2[:, :]
        f1cg_tot = f1cg_tot + dot(h * ecg, cf1w[:, :]) + cf1b[:, :]

    ones_aux = (lax.broadcasted_iota(jnp.int32, (b, 8), 1) == 0).astype(f32)
    rhs = jnp.concatenate([f1cg_tot, ones_aux], axis=1)

    mblk = jnp.broadcast_to(m_ref[:, :], (b, ncg))
    oh = (mblk == lax.broadcasted_iota(jnp.int32, (b, ncg), 1)).astype(f32)
    acc_ref[:, :] += lax.dot_general(
        oh, rhs, dimension_numbers=(((0,), (0,)), ((), ())),
        preferred_element_type=f32)

    @pl.when(pid == nprog - 1)
    def _fin():
        acc = acc_ref[:, :]
        cnt = jnp.maximum(acc[:, F:F + 1], 1.0)
        out_ref[:, :] = acc[:, :F] / cnt


def _node_dense(ES0, ES1, z2, m2, dsqI, deg2, embed_pad, offs, wlist, n, ncg, blk):
    grid = n // blk
    full = lambda shape: pl.BlockSpec(shape, lambda i: tuple(0 for _ in shape))
    wspecs = [full(w.shape) for w in wlist]
    return pl.pallas_call(
        _node_kernel,
        grid=(grid,),
        in_specs=[
            pl.BlockSpec((2, blk, F), lambda i: (0, i, 0)),
            pl.BlockSpec((2, blk, F), lambda i: (0, i, 0)),
            pl.BlockSpec((blk, 1), lambda i: (i, 0)),
            pl.BlockSpec((blk, 1), lambda i: (i, 0)),
            pl.BlockSpec((blk, 1), lambda i: (i, 0)),
            pl.BlockSpec((blk, 1), lambda i: (i, 0)),
            full((F, F)),
            full((1, RBF_PAD)),
        ] + wspecs,
        out_specs=pl.BlockSpec((ncg, F), lambda i: (0, 0)),
        out_shape=jax.ShapeDtypeStruct((ncg, F), jnp.float32),
        scratch_shapes=[pltpu.VMEM((ncg, F + 8), jnp.float32)],
    )(ES0, ES1, z2, m2, dsqI, deg2, embed_pad, offs, *wlist)


def _pad_rows(w):
    return jnp.concatenate([w, jnp.zeros((RBF_PAD - w.shape[0], w.shape[1]), w.dtype)], axis=0)


def kernel(z, xyz, cg_xyz, mapping, nbr_list, atom_embed, params):
    n = xyz.shape[0]
    ncg = cg_xyz.shape[0]
    e = nbr_list.shape[0]
    e_pad = E_PAD

    z = z.astype(jnp.int32)
    mapping = mapping.astype(jnp.int32)
    nbr_list = nbr_list.astype(jnp.int32)

    src = nbr_list[:, 0]
    dst = nbr_list[:, 1]

    xyzf = xyz.astype(jnp.float32)
    cgf = cg_xyz.astype(jnp.float32)
    zero_n = jnp.zeros((N_PAD - n,), jnp.float32)
    xs = jnp.concatenate([xyzf[:, 0], zero_n])
    ys = jnp.concatenate([xyzf[:, 1], zero_n])
    zs = jnp.concatenate([xyzf[:, 2], zero_n])
    zero_c = jnp.zeros((NCG_PAD - ncg,), jnp.float32)
    cgx = jnp.concatenate([cgf[:, 0], zero_c])
    cgy = jnp.concatenate([cgf[:, 1], zero_c])
    cgz = jnp.concatenate([cgf[:, 2], zero_c])
    pad_e0 = jnp.zeros((e_pad - e,), jnp.int32)
    dst_a = jnp.concatenate([dst, pad_e0])
    src_c = jnp.concatenate([src, jnp.full((e_pad - e,), n, jnp.int32)])
    src3d = src_c.reshape(32, _DNC, _DCH)
    mp_pad = jnp.concatenate([mapping, jnp.zeros((N_PAD - n,), jnp.int32)])

    dsq_e, dsq_a = _dsq_sc(xs, ys, zs, cgx, cgy, cgz, dst_a, src3d, mp_pad)
    degp = _deg_sc(src3d)
    dsq_pad = dsq_e.reshape(e_pad // 128, 128)
    deg = (degp[:N_PAD] + degp[N_PAD:])[:n]

    offs = jnp.concatenate([jnp.linspace(0.0, CUTOFF, N_RBF, dtype=jnp.float32),
                            jnp.zeros((RBF_PAD - N_RBF,), jnp.float32)]).reshape(1, RBF_PAD)

    pa0, pa1 = params['atom'][0], params['atom'][1]
    hv0 = _edge_dense(dsq_pad, offs, _pad_rows(pa0['dist_w1']),
                      pa0['dist_b1'].reshape(1, F), e_pad, 8192)
    ES0 = _scatter_sc(hv0.reshape(e_pad, F), src_c)
    hv1 = _edge_dense(dsq_pad, offs, _pad_rows(pa1['dist_w1']),
                      pa1['dist_b1'].reshape(1, F), e_pad, 8192)
    ES1 = _scatter_sc(hv1.reshape(e_pad, F), src_c)

    z2 = z.reshape(n, 1)
    m2 = mapping.reshape(n, 1)
    dsqI2 = dsq_a[:n].reshape(n, 1).astype(jnp.float32)
    deg2 = deg.reshape(n, 1)
    embed_pad = jnp.concatenate(
        [atom_embed, jnp.zeros((F - atom_embed.shape[0], F), jnp.float32)], axis=0)

    wlist = []
    for L in range(2):
        pa = params['atom'][L]
        pc = params['cg'][L]
        wlist += [pa['dist_w2'], pa['dist_b2'].reshape(1, F),
                  pa['l1_w'], pa['l1_b'].reshape(1, F),
                  pa['l2_w'], pa['l2_b'].reshape(1, F),
                  pa['f1_w'], pa['f1_b'].reshape(1, F),
                  _pad_rows(pc['dist_w1']), pc['dist_b1'].reshape(1, F),
                  pc['dist_w2'], pc['dist_b2'].reshape(1, F),
                  pc['f1_w'], pc['f1_b'].reshape(1, F)]

    return _node_dense(ES0, ES1, z2, m2, dsqI2, deg2, embed_pad, offs, wlist, n, ncg, 2000)
